# baseline (speedup 1.0000x reference)
import functools

import jax
import jax.numpy as jnp
from jax import lax
from jax.experimental import pallas as pl
from jax.experimental.pallas import tpu as pltpu
from jax.experimental.pallas import tpu_sc as plsc

_V = 100000
_D = 128
_K = 5
_N = 500000
_NC = 2
_NS = 16
_W = _NC * _NS
_BW = _N // _W
_C = 64
_NCH = 246
_BWP = _NCH * _C
_R = _K + 2

_GDN = lax.GatherDimensionNumbers(
    offset_dims=(), collapsed_slice_dims=(0,), start_index_map=(0,))


def _perm(a, p):
    return lax.gather(a, p[:, None], _GDN, (1,),
                      mode=lax.GatherScatterMode.PROMISE_IN_BOUNDS)
_IC = _R * _C
_OC = 6 * _C


def _sc_dots(embs, idxs):
    mesh = plsc.VectorSubcoreMesh(core_axis_name="c", subcore_axis_name="s")

    @functools.partial(
        pl.kernel,
        out_type=jax.ShapeDtypeStruct((_W * _NCH * _OC,), jnp.float32),
        mesh=mesh,
        compiler_params=pltpu.CompilerParams(needs_layout_passes=False),
        scratch_types=[
            pltpu.VMEM((_IC,), jnp.int32),
            pltpu.VMEM((_IC,), jnp.int32),
            pltpu.VMEM((_IC, _D), jnp.float32),
            pltpu.VMEM((_IC, _D), jnp.float32),
            pltpu.VMEM((_OC,), jnp.float32),
            pltpu.VMEM((_OC,), jnp.float32),
            pltpu.SemaphoreType.DMA,
            pltpu.SemaphoreType.DMA,
            pltpu.SemaphoreType.DMA,
            pltpu.SemaphoreType.DMA,
            pltpu.SemaphoreType.DMA,
        ],
    )
    def body(embs_hbm, idx_hbm, out_hbm, idx_v0, idx_v1, rows_v0, rows_v1,
             out_v0, out_v1, sg0, sg1, si0, si1, so):
        sg = (sg0, sg1)
        si = (si0, si1)
        idxs_v = (idx_v0, idx_v1)
        rows = (rows_v0, rows_v1)
        outs = (out_v0, out_v1)
        wid = lax.axis_index("s") * _NC + lax.axis_index("c")
        ibase = pl.multiple_of(wid * (_NCH + 1) * _IC, 8)
        obase = pl.multiple_of(wid * _NCH * _OC, 8)

        def fire_idx(c, b):
            src = pl.multiple_of(ibase + c * _IC, 8)
            pltpu.async_copy(idx_hbm.at[pl.ds(src, _IC)], idxs_v[b], si[b])

        def wait_idx(b):
            pltpu.make_async_copy(idx_hbm.at[pl.ds(0, _IC)],
                                  idxs_v[b], si[b]).wait()

        def fire_gathers(b):
            for r in range(_R):
                pltpu.async_copy(
                    embs_hbm.at[idxs_v[b].at[pl.ds(r * _C, _C)]],
                    rows[b].at[pl.ds(r * _C, _C)], sg[b])

        def wait_gathers(b):
            pltpu.make_async_copy(embs_hbm.at[pl.ds(0, _IC)],
                                  rows[b], sg[b]).wait()

        def wait_out():
            pltpu.make_async_copy(outs[0],
                                  out_hbm.at[pl.ds(0, _OC)], so).wait()

        def compute(c, b):
            lane = lax.iota(jnp.int32, 16)
            lane0 = lane == 0
            perms = [(lane + r) & 15 for r in (8, 4, 2, 1)]

            def edge(e, carry):
                eu = [rows[b][e, pl.ds(16 * j, 16)] for j in range(8)]
                accs = [eu[0] * rows[b][(1 + s) * _C + e, pl.ds(0, 16)]
                        for s in range(6)]
                for j in range(1, 8):
                    for s in range(6):
                        accs[s] = accs[s] + eu[j] * rows[b][
                            (1 + s) * _C + e, pl.ds(16 * j, 16)]
                for s in range(6):
                    a = accs[s]
                    for p in perms:
                        a = a + _perm(a, p)
                    plsc.store_scatter(
                        outs[b], [jnp.broadcast_to(s * _C + e, (16,))], a,
                        mask=lane0)
                return carry

            lax.fori_loop(0, _C, edge, 0, unroll=2)
            dst = pl.multiple_of(obase + c * _OC, 8)
            pltpu.async_copy(outs[b], out_hbm.at[pl.ds(dst, _OC)], so)

        fire_idx(0, 0)
        wait_idx(0)
        fire_gathers(0)
        fire_idx(1, 1)

        def pair(p, carry):
            for h in (0, 1):
                c = 2 * p + h
                b = h
                wait_gathers(b)

                @pl.when(c < _NCH - 1)
                def _():
                    wait_idx(b ^ 1)
                    fire_gathers(b ^ 1)

                @pl.when(c < _NCH - 2)
                def _():
                    fire_idx(c + 2, b)

                @pl.when(p >= 1)
                def _():
                    wait_out()

                compute(c, b)
            return carry

        lax.fori_loop(0, _NCH // 2, pair, 0)
        wait_out()
        wait_out()

    return body(embs, idxs)


def _tc_loss(dots):
    def body(d_ref, o_ref):
        i = pl.program_id(0)

        @pl.when(i == 0)
        def _():
            o_ref[0, 0] = 0.0

        x = d_ref[0]
        ci = lax.broadcasted_iota(jnp.int32, x.shape, 0)
        li = lax.broadcasted_iota(jnp.int32, x.shape, 1)
        x = jnp.where(li < _C, x, -x)
        t = jnp.exp(-jnp.abs(x))
        s = jnp.where(x >= 0, 1.0 / (1.0 + t), t / (1.0 + t))
        ls = jnp.log(s)
        valid = ci * _C + (li % _C) < _BW
        o_ref[0, 0] += jnp.sum(jnp.where(valid, ls, 0.0)) * (-1.0 / _N)

    return pl.pallas_call(
        body,
        grid=(_W,),
        in_specs=[pl.BlockSpec((1, _NCH, 6 * _C), lambda i: (i, 0, 0))],
        out_shape=jax.ShapeDtypeStruct((1, 1), jnp.float32),
        out_specs=pl.BlockSpec(memory_space=pltpu.SMEM),
    )(dots)


def kernel(input, embs):
    u = input[0]
    v = input[1]
    negs = jax.random.randint(jax.random.key(42), (_N, _K), 0, _V)
    pad = _BWP - _BW
    up = jnp.pad(u.reshape(_W, _BW), ((0, 0), (0, pad)))
    vp = jnp.pad(v.reshape(_W, _BW), ((0, 0), (0, pad)))
    negsp = jnp.pad(negs.T.reshape(_K, _W, _BW), ((0, 0), (0, 0), (0, pad)))
    blocks = jnp.concatenate(
        [up.reshape(_W, 1, _NCH, _C), vp.reshape(_W, 1, _NCH, _C),
         negsp.reshape(_K, _W, _NCH, _C).transpose(1, 0, 2, 3)], axis=1)
    blocks = blocks.transpose(0, 2, 1, 3)
    blocks = jnp.pad(blocks, ((0, 0), (0, 1), (0, 0), (0, 0)))
    dots = _sc_dots(embs, blocks.reshape(-1))
    return _tc_loss(dots.reshape(_W, _NCH, 6 * _C))[0, 0]
